# baseline (speedup 1.0000x reference)
import functools

import jax
import jax.numpy as jnp
from jax import lax
from jax.experimental import pallas as pl
from jax.experimental.pallas import tpu as pltpu
from jax.experimental.pallas import tpu_sc as plsc

H = 128
CH = 128
NW = 32
BN = 1000
BE = 2000


def _prelu(x, a):
    return jnp.where(x >= 0, x, a * x)


def _node_mlp_body(ns_ref, nv_ref, w1c_ref, b1c_ref, w2c_ref, b2c_ref,
                   w1v_ref, b1v_ref, w2v_ref, b2v_ref, a_ref,
                   outc_ref, outv_ref):
    x = ns_ref[...]
    nv = nv_ref[...]
    for (w1, b1, w2, b2, ia, out) in (
        (w1c_ref, b1c_ref, w2c_ref, b2c_ref, 0, outc_ref),
        (w1v_ref, b1v_ref, w2v_ref, b2v_ref, 2, outv_ref),
    ):
        h = lax.dot_general(x, w1[...], (((1,), (1,)), ((), ())),
                            preferred_element_type=jnp.float32)
        h = _prelu(h + b1[...], a_ref[ia])
        o = lax.dot_general(h, w2[...], (((1,), (1,)), ((), ())),
                            preferred_element_type=jnp.float32)
        out[:, :3 * H] = _prelu(o + b2[...], a_ref[ia + 1])
        out[:, 3 * H:] = nv


def _node_mlp(node_scalar, tab_nv, w1c, b1c, w2c, b2c, w1v, b1v, w2v, b2v,
              avec):
    n = node_scalar.shape[0]
    grid = n // BN
    full = lambda shape: pl.BlockSpec(shape, lambda i: (0, 0))
    return pl.pallas_call(
        _node_mlp_body,
        grid=(grid,),
        in_specs=[
            pl.BlockSpec((BN, H), lambda i: (i, 0)),
            pl.BlockSpec((BN, 3 * H), lambda i: (i, 0)),
            full((H, H)), full((1, H)), full((3 * H, H)), full((1, 3 * H)),
            full((H, H)), full((1, H)), full((3 * H, H)), full((1, 3 * H)),
            pl.BlockSpec(memory_space=pltpu.SMEM),
        ],
        out_specs=[
            pl.BlockSpec((BN, 6 * H), lambda i: (i, 0)),
            pl.BlockSpec((BN, 6 * H), lambda i: (i, 0)),
        ],
        out_shape=[
            jax.ShapeDtypeStruct((n, 6 * H), jnp.float32),
            jax.ShapeDtypeStruct((n, 6 * H), jnp.float32),
        ],
    )(node_scalar, tab_nv, w1c, b1c.reshape(1, H), w2c, b2c.reshape(1, 3 * H),
      w1v, b1v.reshape(1, H), w2v, b2v.reshape(1, 3 * H), avec)


GCH = 128


def _gather_body(nchunk, quota, tab_ref, src_ref, g_ref,
                 idx_ref, buf0_ref, buf1_ref, sem0, sem1):
    wid = lax.axis_index("s") * 2 + lax.axis_index("c")
    start = wid * quota
    cnt = jnp.clip(nchunk - start, 0, quota)
    bufs = (buf0_ref, buf1_ref)
    sems = (sem0, sem1)

    pltpu.sync_copy(src_ref.at[pl.ds(start * GCH, quota * GCH)], idx_ref)

    def _start(j, b):
        idx = idx_ref.at[pl.ds(j * GCH, GCH)]
        pltpu.async_copy(tab_ref.at[idx], bufs[b], sems[b])

    for b in (0, 1):
        @pl.when(b < cnt)
        def _(b=b):
            _start(b, b)

    def body(g, carry):
        for b in (0, 1):
            j = g * 2 + b

            @pl.when(j < cnt)
            def _(j=j, b=b):
                pltpu.make_async_copy(
                    tab_ref.at[idx_ref.at[pl.ds(0, GCH)]],
                    bufs[b], sems[b]).wait()
                pltpu.sync_copy(bufs[b],
                                g_ref.at[pl.ds((start + j) * GCH, GCH)])

            @pl.when(j + 2 < cnt)
            def _(j=j, b=b):
                _start(j + 2, b)
        return carry

    lax.fori_loop(0, quota // 2, body, 0)


def _quota(nchunk):
    q = -(-nchunk // NW)
    return -(-q // 8) * 8


def _sc_gather(tab, src, e):
    nchunk = e // GCH
    quota = _quota(nchunk)
    mesh = plsc.VectorSubcoreMesh(core_axis_name="c", subcore_axis_name="s")
    kern = pl.kernel(
        functools.partial(_gather_body, nchunk, quota),
        out_type=jax.ShapeDtypeStruct((e, 3 * H), jnp.float32),
        mesh=mesh,
        scratch_types=[
            pltpu.VMEM((quota * GCH,), jnp.int32),
            pltpu.VMEM((GCH, 3 * H), jnp.float32),
            pltpu.VMEM((GCH, 3 * H), jnp.float32),
            pltpu.SemaphoreType.DMA,
            pltpu.SemaphoreType.DMA,
        ],
    )
    return kern(tab, src)


def _edge_body(rbf_ref, aux_ref, g_ref,
               w1_ref, b1_ref, w2_ref, b2_ref, a_ref,
               ms_ref, mv0_ref, mv1_ref, mv2_ref):
    x = rbf_ref[...]
    h = lax.dot_general(x, w1_ref[...], (((1,), (1,)), ((), ())),
                        preferred_element_type=jnp.float32)
    h = _prelu(h + b1_ref[...], a_ref[0])
    fw = lax.dot_general(h, w2_ref[...], (((1,), (1,)), ((), ())),
                         preferred_element_type=jnp.float32)
    fw = _prelu(fw + b2_ref[...], a_ref[1])
    wi = lax.bitcast_convert_type(g_ref[...], jnp.int32)
    gs = lax.bitcast_convert_type(wi << 16, jnp.float32)
    gv = lax.bitcast_convert_type(wi & jnp.int32(-65536), jnp.float32)
    fo = gs * fw
    gate = fo[:, :H]
    ge = fo[:, 2 * H:]
    aux = aux_ref[...]
    rinv = 1.0 / aux[:, 3:4]
    ms_ref[...] = fo[:, H:2 * H]
    for k, mv in ((0, mv0_ref), (1, mv1_ref), (2, mv2_ref)):
        mv[...] = (gv[:, k * H:(k + 1) * H] * gate
                   + (aux[:, k:k + 1] * rinv) * ge)


def _edge_stage(rbf_pad, aux, g, w1p, b1, w2, b2, avec, e):
    grid = e // BE
    full = lambda shape: pl.BlockSpec(shape, lambda i: (0, 0))
    blk = lambda w: pl.BlockSpec((BE, w), lambda i: (i, 0))
    o = jax.ShapeDtypeStruct((e, H), jnp.float32)
    return pl.pallas_call(
        _edge_body,
        grid=(grid,),
        in_specs=[
            blk(32), blk(8), blk(3 * H),
            full((H, 32)), full((1, H)), full((3 * H, H)), full((1, 3 * H)),
            pl.BlockSpec(memory_space=pltpu.SMEM),
        ],
        out_specs=[blk(H), blk(H), blk(H), blk(H)],
        out_shape=[o, o, o, o],
    )(rbf_pad, aux, g, w1p, b1.reshape(1, H), w2, b2.reshape(1, 3 * H), avec)


def _scatter_body(nchunk, n, quota,
                  mcs_ref, mc0_ref, mc1_ref, mc2_ref,
                  mvs_ref, mv0_ref, mv1_ref, mv2_ref,
                  dstc_ref, dstv_ref, zeros_ref, out_ref,
                  idx0_ref, idx1_ref, msg0_ref, msg1_ref, w_ref,
                  acc_ref, msem0, msem1, isem0, isem1):
    cid = lax.axis_index("c")
    tid = lax.axis_index("s")
    wid = tid * 2 + cid
    start = wid * quota
    cnt = jnp.clip(nchunk - start, 0, quota)
    zrows = 80
    nzchunk = n // zrows
    ziters = -(-nzchunk // 16)

    mbufs = (msg0_ref, msg1_ref)
    ibufs = (idx0_ref, idx1_ref)
    msems = (msem0, msem1)
    isems = (isem0, isem1)

    passes = ((mcs_ref, mvs_ref), (mc0_ref, mv0_ref),
              (mc1_ref, mv1_ref), (mc2_ref, mv2_ref))
    for p, (msg_che, msg_vdw) in enumerate(passes):
        pltpu.sync_copy(zeros_ref, w_ref)
        for zi in range(ziters):
            c = zi * 16 + tid

            @pl.when(c < nzchunk)
            def _():
                pltpu.sync_copy(w_ref, acc_ref.at[pl.ds(c * zrows, zrows)])
        plsc.subcore_barrier()
        for msg, dref in ((msg_che, dstc_ref), (msg_vdw, dstv_ref)):
            def _start(j, b, msg=msg, dref=dref):
                base = (start + j) * CH
                pltpu.async_copy(msg.at[pl.ds(base, CH)], mbufs[b], msems[b])
                pltpu.async_copy(dref.at[pl.ds(base, CH)], ibufs[b], isems[b])

            for b in (0, 1):
                @pl.when(b < cnt)
                def _(b=b):
                    _start(b, b)

            def body(g, carry, msg=msg, dref=dref, _start=_start):
                for b in (0, 1):
                    j = g * 2 + b

                    @pl.when(j < cnt)
                    def _(j=j, b=b):
                        pltpu.make_async_copy(
                            msg.at[pl.ds(0, CH)], mbufs[b], msems[b]).wait()
                        pltpu.make_async_copy(
                            dref.at[pl.ds(0, CH)], ibufs[b], isems[b]).wait()
                        pltpu.sync_copy(mbufs[b], acc_ref.at[ibufs[b]],
                                        add=True)

                    @pl.when(j + 2 < cnt)
                    def _(j=j, b=b):
                        _start(j + 2, b)
                return carry

            lax.fori_loop(0, quota // 2, body, 0)
        plsc.subcore_barrier()
        obase = (p * 2 + cid) * n
        for zi in range(ziters):
            c = zi * 16 + tid

            @pl.when(c < nzchunk)
            def _():
                pltpu.sync_copy(acc_ref.at[pl.ds(c * zrows, zrows)], w_ref)
                pltpu.sync_copy(w_ref, out_ref.at[pl.ds(obase + c * zrows, zrows)])
        plsc.subcore_barrier()


def _sc_scatter(msgs_che, msgs_vdw, dst_che, dst_vdw, zeros, e, n):
    nchunk = e // CH
    quota = _quota(nchunk)
    mesh = plsc.VectorSubcoreMesh(core_axis_name="c", subcore_axis_name="s")
    kern = pl.kernel(
        functools.partial(_scatter_body, nchunk, n, quota),
        out_type=jax.ShapeDtypeStruct((8 * n, H), jnp.float32),
        mesh=mesh,
        scratch_types=[
            pltpu.VMEM((CH,), jnp.int32),
            pltpu.VMEM((CH,), jnp.int32),
            pltpu.VMEM((CH, H), jnp.float32),
            pltpu.VMEM((CH, H), jnp.float32),
            pltpu.VMEM((80, H), jnp.float32),
            pltpu.VMEM_SHARED((n, H), jnp.float32),
            pltpu.SemaphoreType.DMA,
            pltpu.SemaphoreType.DMA,
            pltpu.SemaphoreType.DMA,
            pltpu.SemaphoreType.DMA,
        ],
    )
    return kern(*msgs_che, *msgs_vdw, dst_che, dst_vdw, zeros)


def _combine_body(ns_ref, nv0_ref, nv1_ref, nv2_ref,
                  ps0_ref, ps1_ref, p00_ref, p01_ref,
                  p10_ref, p11_ref, p20_ref, p21_ref,
                  os_ref, ov0_ref, ov1_ref, ov2_ref):
    os_ref[...] = ns_ref[...] + ps0_ref[...] + ps1_ref[...]
    ov0_ref[...] = nv0_ref[...] + p00_ref[...] + p01_ref[...]
    ov1_ref[...] = nv1_ref[...] + p10_ref[...] + p11_ref[...]
    ov2_ref[...] = nv2_ref[...] + p20_ref[...] + p21_ref[...]


def _combine(node_scalar, nv0, nv1, nv2, part, n):
    grid = n // BN
    nb = n // BN
    blk = pl.BlockSpec((BN, H), lambda i: (i, 0))

    def pblk(p, c):
        off = (p * 2 + c) * nb
        return pl.BlockSpec((BN, H), lambda i, off=off: (off + i, 0))

    o = jax.ShapeDtypeStruct((n, H), jnp.float32)
    return pl.pallas_call(
        _combine_body,
        grid=(grid,),
        in_specs=[blk, blk, blk, blk,
                  pblk(0, 0), pblk(0, 1), pblk(1, 0), pblk(1, 1),
                  pblk(2, 0), pblk(2, 1), pblk(3, 0), pblk(3, 1)],
        out_specs=[blk, blk, blk, blk],
        out_shape=[o, o, o, o],
    )(node_scalar, nv0, nv1, nv2,
      part, part, part, part, part, part, part, part)


def kernel(node_scalar, node_vector, che_edge, che_edge_diff, che_edge_dist,
           che_rbf_dist, vdw_edge, vdw_edge_diff, vdw_edge_dist, vdw_rbf_dist,
           che_s_W1, che_s_b1, che_s_a1, che_s_W2, che_s_b2, che_s_a2,
           che_f_W1, che_f_b1, che_f_a1, che_f_W2, che_f_b2, che_f_a2,
           vdw_s_W1, vdw_s_b1, vdw_s_a1, vdw_s_W2, vdw_s_b2, vdw_s_a2,
           vdw_f_W1, vdw_f_b1, vdw_f_a1, vdw_f_W2, vdw_f_b2, vdw_f_a2):
    n = node_scalar.shape[0]
    e = che_edge.shape[0]
    pad_e = NW * _quota(e // CH) * CH
    pad_g = NW * _quota(e // GCH) * GCH

    a_s = jnp.stack([che_s_a1, che_s_a2, vdw_s_a1, vdw_s_a2])
    tab_nv = node_vector.reshape(n, 3 * H)
    tab_che, tab_vdw = _node_mlp(node_scalar, tab_nv, che_s_W1, che_s_b1,
                                 che_s_W2, che_s_b2, vdw_s_W1, vdw_s_b1,
                                 vdw_s_W2, vdw_s_b2, a_s)

    def _pack(tab):
        t16 = tab.astype(jnp.bfloat16)
        t2 = jnp.stack([t16[:, :3 * H], t16[:, 3 * H:]], axis=-1)
        return lax.bitcast_convert_type(t2, jnp.float32)

    src_che = jnp.pad(che_edge[:, 1], (0, pad_g - e))
    src_vdw = jnp.pad(vdw_edge[:, 1], (0, pad_g - e))
    g_che = _sc_gather(_pack(tab_che), src_che, e)
    g_vdw = _sc_gather(_pack(tab_vdw), src_vdw, e)

    a_fc = jnp.stack([che_f_a1, che_f_a2])
    a_fv = jnp.stack([vdw_f_a1, vdw_f_a2])
    ee = che_rbf_dist.shape[1]
    rbf_che = jnp.pad(che_rbf_dist, ((0, 0), (0, 32 - ee)))
    rbf_vdw = jnp.pad(vdw_rbf_dist, ((0, 0), (0, 32 - ee)))
    w1c = jnp.pad(che_f_W1, ((0, 0), (0, 32 - ee)))
    w1v = jnp.pad(vdw_f_W1, ((0, 0), (0, 32 - ee)))
    aux_che = jnp.pad(
        jnp.concatenate([che_edge_diff, che_edge_dist[:, None]], axis=1),
        ((0, 0), (0, 4)))
    aux_vdw = jnp.pad(
        jnp.concatenate([vdw_edge_diff, vdw_edge_dist[:, None]], axis=1),
        ((0, 0), (0, 4)))
    msgs_che = _edge_stage(rbf_che, aux_che, g_che,
                           w1c, che_f_b1, che_f_W2, che_f_b2, a_fc, e)
    msgs_vdw = _edge_stage(rbf_vdw, aux_vdw, g_vdw,
                           w1v, vdw_f_b1, vdw_f_W2, vdw_f_b2, a_fv, e)

    dst_che = jnp.pad(che_edge[:, 0], (0, pad_e - e))
    dst_vdw = jnp.pad(vdw_edge[:, 0], (0, pad_e - e))
    zeros = jnp.zeros((80, H), jnp.float32)
    part = _sc_scatter(msgs_che, msgs_vdw, dst_che, dst_vdw, zeros, e, n)

    nv0 = node_vector[:, 0, :]
    nv1 = node_vector[:, 1, :]
    nv2 = node_vector[:, 2, :]
    os_, ov0, ov1, ov2 = _combine(node_scalar, nv0, nv1, nv2, part, n)
    return os_, jnp.stack([ov0, ov1, ov2], axis=1)


# speedup vs baseline: 1.0554x; 1.0554x over previous
import functools

import jax
import jax.numpy as jnp
from jax import lax
from jax.experimental import pallas as pl
from jax.experimental.pallas import tpu as pltpu
from jax.experimental.pallas import tpu_sc as plsc

H = 128
CH = 128
NW = 32
BN = 1000
BE = 2000


def _prelu(x, a):
    return jnp.where(x >= 0, x, a * x)


def _node_mlp_body(ns_ref, nv_ref, w1c_ref, b1c_ref, w2c_ref, b2c_ref,
                   w1v_ref, b1v_ref, w2v_ref, b2v_ref, a_ref,
                   outc_ref, outv_ref):
    x = ns_ref[...]
    nv = nv_ref[...]
    for (w1, b1, w2, b2, ia, out) in (
        (w1c_ref, b1c_ref, w2c_ref, b2c_ref, 0, outc_ref),
        (w1v_ref, b1v_ref, w2v_ref, b2v_ref, 2, outv_ref),
    ):
        h = lax.dot_general(x, w1[...], (((1,), (1,)), ((), ())),
                            preferred_element_type=jnp.float32)
        h = _prelu(h + b1[...], a_ref[ia])
        o = lax.dot_general(h, w2[...], (((1,), (1,)), ((), ())),
                            preferred_element_type=jnp.float32)
        out[:, :3 * H] = _prelu(o + b2[...], a_ref[ia + 1])
        out[:, 3 * H:] = nv


def _node_mlp(node_scalar, tab_nv, w1c, b1c, w2c, b2c, w1v, b1v, w2v, b2v,
              avec):
    n = node_scalar.shape[0]
    grid = n // BN
    full = lambda shape: pl.BlockSpec(shape, lambda i: (0, 0))
    return pl.pallas_call(
        _node_mlp_body,
        grid=(grid,),
        in_specs=[
            pl.BlockSpec((BN, H), lambda i: (i, 0)),
            pl.BlockSpec((BN, 3 * H), lambda i: (i, 0)),
            full((H, H)), full((1, H)), full((3 * H, H)), full((1, 3 * H)),
            full((H, H)), full((1, H)), full((3 * H, H)), full((1, 3 * H)),
            pl.BlockSpec(memory_space=pltpu.SMEM),
        ],
        out_specs=[
            pl.BlockSpec((BN, 6 * H), lambda i: (i, 0)),
            pl.BlockSpec((BN, 6 * H), lambda i: (i, 0)),
        ],
        out_shape=[
            jax.ShapeDtypeStruct((n, 6 * H), jnp.float32),
            jax.ShapeDtypeStruct((n, 6 * H), jnp.float32),
        ],
    )(node_scalar, tab_nv, w1c, b1c.reshape(1, H), w2c, b2c.reshape(1, 3 * H),
      w1v, b1v.reshape(1, H), w2v, b2v.reshape(1, 3 * H), avec)


GCH = 128


def _gather_body(nchunk, quota, tab_ref, src_ref, g_ref,
                 idx_ref, buf0_ref, buf1_ref, sem0, sem1):
    wid = lax.axis_index("s") * 2 + lax.axis_index("c")
    start = wid * quota
    cnt = jnp.clip(nchunk - start, 0, quota)
    bufs = (buf0_ref, buf1_ref)
    sems = (sem0, sem1)

    pltpu.sync_copy(src_ref.at[pl.ds(start * GCH, quota * GCH)], idx_ref)

    def _start(j, b):
        idx = idx_ref.at[pl.ds(j * GCH, GCH)]
        pltpu.async_copy(tab_ref.at[idx], bufs[b], sems[b])

    for b in (0, 1):
        @pl.when(b < cnt)
        def _(b=b):
            _start(b, b)

    def body(g, carry):
        for b in (0, 1):
            j = g * 2 + b

            @pl.when(j < cnt)
            def _(j=j, b=b):
                pltpu.make_async_copy(
                    tab_ref.at[idx_ref.at[pl.ds(0, GCH)]],
                    bufs[b], sems[b]).wait()
                pltpu.sync_copy(bufs[b],
                                g_ref.at[pl.ds((start + j) * GCH, GCH)])

            @pl.when(j + 2 < cnt)
            def _(j=j, b=b):
                _start(j + 2, b)
        return carry

    lax.fori_loop(0, quota // 2, body, 0)


def _quota(nchunk):
    q = -(-nchunk // NW)
    return -(-q // 8) * 8


def _sc_gather(tab, src, e):
    nchunk = e // GCH
    quota = _quota(nchunk)
    mesh = plsc.VectorSubcoreMesh(core_axis_name="c", subcore_axis_name="s")
    kern = pl.kernel(
        functools.partial(_gather_body, nchunk, quota),
        out_type=jax.ShapeDtypeStruct((e, 3 * H), jnp.float32),
        mesh=mesh,
        scratch_types=[
            pltpu.VMEM((quota * GCH,), jnp.int32),
            pltpu.VMEM((GCH, 3 * H), jnp.float32),
            pltpu.VMEM((GCH, 3 * H), jnp.float32),
            pltpu.SemaphoreType.DMA,
            pltpu.SemaphoreType.DMA,
        ],
    )
    return kern(tab, src)


def _edge_body(rbf_ref, aux_ref, g_ref,
               w1_ref, b1_ref, w2_ref, b2_ref, a_ref,
               ms_ref, mv0_ref, mv1_ref, mv2_ref):
    x = rbf_ref[...]
    h = lax.dot_general(x, w1_ref[...], (((1,), (1,)), ((), ())),
                        preferred_element_type=jnp.float32)
    h = _prelu(h + b1_ref[...], a_ref[0])
    fw = lax.dot_general(h, w2_ref[...], (((1,), (1,)), ((), ())),
                         preferred_element_type=jnp.float32)
    fw = _prelu(fw + b2_ref[...], a_ref[1])
    wi = lax.bitcast_convert_type(g_ref[...], jnp.int32)
    gs = lax.bitcast_convert_type(wi << 16, jnp.float32)
    gv = lax.bitcast_convert_type(wi & jnp.int32(-65536), jnp.float32)
    fo = gs * fw
    gate = fo[:, :H]
    ge = fo[:, 2 * H:]
    aux = aux_ref[...]
    rinv = 1.0 / aux[:, 3:4]
    ms_ref[...] = fo[:, H:2 * H]
    for k, mv in ((0, mv0_ref), (1, mv1_ref), (2, mv2_ref)):
        mv[...] = (gv[:, k * H:(k + 1) * H] * gate
                   + (aux[:, k:k + 1] * rinv) * ge)


def _edge_stage(rbf_pad, aux, g, w1p, b1, w2, b2, avec, e):
    grid = e // BE
    full = lambda shape: pl.BlockSpec(shape, lambda i: (0, 0))
    blk = lambda w: pl.BlockSpec((BE, w), lambda i: (i, 0))
    o = jax.ShapeDtypeStruct((e, H), jnp.float32)
    return pl.pallas_call(
        _edge_body,
        grid=(grid,),
        in_specs=[
            blk(32), blk(8), blk(3 * H),
            full((H, 32)), full((1, H)), full((3 * H, H)), full((1, 3 * H)),
            pl.BlockSpec(memory_space=pltpu.SMEM),
        ],
        out_specs=[blk(H), blk(H), blk(H), blk(H)],
        out_shape=[o, o, o, o],
    )(rbf_pad, aux, g, w1p, b1.reshape(1, H), w2, b2.reshape(1, 3 * H), avec)


def _scatter_body(nchunk, n, quota,
                  ms_ref, m0_ref, m1_ref, m2_ref,
                  dst_ref, zeros_ref, out_ref,
                  idx0_ref, idx1_ref, msg0_ref, msg1_ref, w_ref,
                  acc_ref, msem0, msem1, isem0, isem1):
    cid = lax.axis_index("c")
    tid = lax.axis_index("s")
    wid = tid * 2 + cid
    start = wid * quota
    cnt = jnp.clip(nchunk - start, 0, quota)
    zrows = 80
    nzchunk = n // zrows
    ziters = -(-nzchunk // 16)

    mbufs = (msg0_ref, msg1_ref)
    ibufs = (idx0_ref, idx1_ref)
    msems = (msem0, msem1)
    isems = (isem0, isem1)

    for p, msg in enumerate((ms_ref, m0_ref, m1_ref, m2_ref)):
        pltpu.sync_copy(zeros_ref, w_ref)
        for zi in range(ziters):
            c = zi * 16 + tid

            @pl.when(c < nzchunk)
            def _():
                pltpu.sync_copy(w_ref, acc_ref.at[pl.ds(c * zrows, zrows)])
        plsc.subcore_barrier()

        def _start(j, b, msg=msg):
            base = (start + j) * CH
            pltpu.async_copy(msg.at[pl.ds(base, CH)], mbufs[b], msems[b])
            pltpu.async_copy(dst_ref.at[pl.ds(base, CH)], ibufs[b], isems[b])

        for b in (0, 1):
            @pl.when(b < cnt)
            def _(b=b):
                _start(b, b)

        def body(g, carry, msg=msg, _start=_start):
            for b in (0, 1):
                j = g * 2 + b

                @pl.when(j < cnt)
                def _(j=j, b=b):
                    pltpu.make_async_copy(
                        msg.at[pl.ds(0, CH)], mbufs[b], msems[b]).wait()
                    pltpu.make_async_copy(
                        dst_ref.at[pl.ds(0, CH)], ibufs[b], isems[b]).wait()
                    pltpu.sync_copy(mbufs[b], acc_ref.at[ibufs[b]],
                                    add=True)

                @pl.when(j + 2 < cnt)
                def _(j=j, b=b):
                    _start(j + 2, b)
            return carry

        lax.fori_loop(0, quota // 2, body, 0)
        plsc.subcore_barrier()
        obase = (p * 2 + cid) * n
        for zi in range(ziters):
            c = zi * 16 + tid

            @pl.when(c < nzchunk)
            def _():
                pltpu.sync_copy(acc_ref.at[pl.ds(c * zrows, zrows)], w_ref)
                pltpu.sync_copy(w_ref, out_ref.at[pl.ds(obase + c * zrows, zrows)])
        plsc.subcore_barrier()


def _sc_scatter(msgs, dst, zeros, e, n):
    nchunk = e // CH
    quota = _quota(nchunk)
    mesh = plsc.VectorSubcoreMesh(core_axis_name="c", subcore_axis_name="s")
    kern = pl.kernel(
        functools.partial(_scatter_body, nchunk, n, quota),
        out_type=jax.ShapeDtypeStruct((8 * n, H), jnp.float32),
        mesh=mesh,
        scratch_types=[
            pltpu.VMEM((CH,), jnp.int32),
            pltpu.VMEM((CH,), jnp.int32),
            pltpu.VMEM((CH, H), jnp.float32),
            pltpu.VMEM((CH, H), jnp.float32),
            pltpu.VMEM((80, H), jnp.float32),
            pltpu.VMEM_SHARED((n, H), jnp.float32),
            pltpu.SemaphoreType.DMA,
            pltpu.SemaphoreType.DMA,
            pltpu.SemaphoreType.DMA,
            pltpu.SemaphoreType.DMA,
        ],
    )
    return kern(*msgs, dst, zeros)


def _combine_body(*refs):
    base = refs[:4]
    pa = refs[4:12]
    pb = refs[12:20]
    outs = refs[20:]
    for k in range(4):
        outs[k][...] = (base[k][...]
                        + pa[2 * k][...] + pa[2 * k + 1][...]
                        + pb[2 * k][...] + pb[2 * k + 1][...])


def _combine(node_scalar, nv0, nv1, nv2, part_a, part_b, n):
    grid = n // BN
    nb = n // BN
    blk = pl.BlockSpec((BN, H), lambda i: (i, 0))

    def pblk(p, c):
        off = (p * 2 + c) * nb
        return pl.BlockSpec((BN, H), lambda i, off=off: (off + i, 0))

    pblks = [pblk(p, c) for p in range(4) for c in range(2)]
    o = jax.ShapeDtypeStruct((n, H), jnp.float32)
    return pl.pallas_call(
        _combine_body,
        grid=(grid,),
        in_specs=[blk, blk, blk, blk] + pblks + pblks,
        out_specs=[blk, blk, blk, blk],
        out_shape=[o, o, o, o],
    )(node_scalar, nv0, nv1, nv2,
      *([part_a] * 8), *([part_b] * 8))


def kernel(node_scalar, node_vector, che_edge, che_edge_diff, che_edge_dist,
           che_rbf_dist, vdw_edge, vdw_edge_diff, vdw_edge_dist, vdw_rbf_dist,
           che_s_W1, che_s_b1, che_s_a1, che_s_W2, che_s_b2, che_s_a2,
           che_f_W1, che_f_b1, che_f_a1, che_f_W2, che_f_b2, che_f_a2,
           vdw_s_W1, vdw_s_b1, vdw_s_a1, vdw_s_W2, vdw_s_b2, vdw_s_a2,
           vdw_f_W1, vdw_f_b1, vdw_f_a1, vdw_f_W2, vdw_f_b2, vdw_f_a2):
    n = node_scalar.shape[0]
    e = che_edge.shape[0]
    pad_e = NW * _quota(e // CH) * CH
    pad_g = NW * _quota(e // GCH) * GCH

    a_s = jnp.stack([che_s_a1, che_s_a2, vdw_s_a1, vdw_s_a2])
    tab_nv = node_vector.reshape(n, 3 * H)
    tab_che, tab_vdw = _node_mlp(node_scalar, tab_nv, che_s_W1, che_s_b1,
                                 che_s_W2, che_s_b2, vdw_s_W1, vdw_s_b1,
                                 vdw_s_W2, vdw_s_b2, a_s)

    def _pack(tab):
        t16 = tab.astype(jnp.bfloat16)
        t2 = jnp.stack([t16[:, :3 * H], t16[:, 3 * H:]], axis=-1)
        return lax.bitcast_convert_type(t2, jnp.float32)

    src_che = jnp.pad(che_edge[:, 1], (0, pad_g - e))
    src_vdw = jnp.pad(vdw_edge[:, 1], (0, pad_g - e))
    g_che = _sc_gather(_pack(tab_che), src_che, e)
    g_vdw = _sc_gather(_pack(tab_vdw), src_vdw, e)

    a_fc = jnp.stack([che_f_a1, che_f_a2])
    a_fv = jnp.stack([vdw_f_a1, vdw_f_a2])
    ee = che_rbf_dist.shape[1]
    rbf_che = jnp.pad(che_rbf_dist, ((0, 0), (0, 32 - ee)))
    rbf_vdw = jnp.pad(vdw_rbf_dist, ((0, 0), (0, 32 - ee)))
    w1c = jnp.pad(che_f_W1, ((0, 0), (0, 32 - ee)))
    w1v = jnp.pad(vdw_f_W1, ((0, 0), (0, 32 - ee)))
    aux_che = jnp.pad(
        jnp.concatenate([che_edge_diff, che_edge_dist[:, None]], axis=1),
        ((0, 0), (0, 4)))
    aux_vdw = jnp.pad(
        jnp.concatenate([vdw_edge_diff, vdw_edge_dist[:, None]], axis=1),
        ((0, 0), (0, 4)))
    msgs_che = _edge_stage(rbf_che, aux_che, g_che,
                           w1c, che_f_b1, che_f_W2, che_f_b2, a_fc, e)
    msgs_vdw = _edge_stage(rbf_vdw, aux_vdw, g_vdw,
                           w1v, vdw_f_b1, vdw_f_W2, vdw_f_b2, a_fv, e)

    dst_che = jnp.pad(che_edge[:, 0], (0, pad_e - e))
    dst_vdw = jnp.pad(vdw_edge[:, 0], (0, pad_e - e))
    zeros = jnp.zeros((80, H), jnp.float32)
    part_che = _sc_scatter(msgs_che, dst_che, zeros, e, n)
    part_vdw = _sc_scatter(msgs_vdw, dst_vdw, zeros, e, n)

    nv0 = node_vector[:, 0, :]
    nv1 = node_vector[:, 1, :]
    nv2 = node_vector[:, 2, :]
    os_, ov0, ov1, ov2 = _combine(node_scalar, nv0, nv1, nv2,
                                  part_che, part_vdw, n)
    return os_, jnp.stack([ov0, ov1, ov2], axis=1)


# speedup vs baseline: 1.0561x; 1.0007x over previous
import functools

import jax
import jax.numpy as jnp
from jax import lax
from jax.experimental import pallas as pl
from jax.experimental.pallas import tpu as pltpu
from jax.experimental.pallas import tpu_sc as plsc

H = 128
CH = 128
NW = 32
BN = 1000
BE = 2000


def _prelu(x, a):
    return jnp.where(x >= 0, x, a * x)


def _node_mlp_body(ns_ref, nv_ref, w1c_ref, b1c_ref, w2c_ref, b2c_ref,
                   w1v_ref, b1v_ref, w2v_ref, b2v_ref, a_ref,
                   outc_ref, outv_ref):
    x = ns_ref[...]
    nv = nv_ref[...]
    for (w1, b1, w2, b2, ia, out) in (
        (w1c_ref, b1c_ref, w2c_ref, b2c_ref, 0, outc_ref),
        (w1v_ref, b1v_ref, w2v_ref, b2v_ref, 2, outv_ref),
    ):
        h = lax.dot_general(x, w1[...], (((1,), (1,)), ((), ())),
                            preferred_element_type=jnp.float32)
        h = _prelu(h + b1[...], a_ref[ia])
        o = lax.dot_general(h, w2[...], (((1,), (1,)), ((), ())),
                            preferred_element_type=jnp.float32)
        out[:, :3 * H] = _prelu(o + b2[...], a_ref[ia + 1])
        out[:, 3 * H:] = nv


def _node_mlp(node_scalar, tab_nv, w1c, b1c, w2c, b2c, w1v, b1v, w2v, b2v,
              avec):
    n = node_scalar.shape[0]
    grid = n // BN
    full = lambda shape: pl.BlockSpec(shape, lambda i: (0, 0))
    return pl.pallas_call(
        _node_mlp_body,
        grid=(grid,),
        in_specs=[
            pl.BlockSpec((BN, H), lambda i: (i, 0)),
            pl.BlockSpec((BN, 3 * H), lambda i: (i, 0)),
            full((H, H)), full((1, H)), full((3 * H, H)), full((1, 3 * H)),
            full((H, H)), full((1, H)), full((3 * H, H)), full((1, 3 * H)),
            pl.BlockSpec(memory_space=pltpu.SMEM),
        ],
        out_specs=[
            pl.BlockSpec((BN, 6 * H), lambda i: (i, 0)),
            pl.BlockSpec((BN, 6 * H), lambda i: (i, 0)),
        ],
        out_shape=[
            jax.ShapeDtypeStruct((n, 6 * H), jnp.float32),
            jax.ShapeDtypeStruct((n, 6 * H), jnp.float32),
        ],
    )(node_scalar, tab_nv, w1c, b1c.reshape(1, H), w2c, b2c.reshape(1, 3 * H),
      w1v, b1v.reshape(1, H), w2v, b2v.reshape(1, 3 * H), avec)


GCH = 128


def _gather_body(nchunk, quota, tab_ref, src_ref, g_ref,
                 idx_ref, buf0_ref, buf1_ref, sem0, sem1):
    wid = lax.axis_index("s") * 2 + lax.axis_index("c")
    start = wid * quota
    cnt = jnp.clip(nchunk - start, 0, quota)
    bufs = (buf0_ref, buf1_ref)
    sems = (sem0, sem1)

    pltpu.sync_copy(src_ref.at[pl.ds(start * GCH, quota * GCH)], idx_ref)

    def _start(j, b):
        idx = idx_ref.at[pl.ds(j * GCH, GCH)]
        pltpu.async_copy(tab_ref.at[idx], bufs[b], sems[b])

    for b in (0, 1):
        @pl.when(b < cnt)
        def _(b=b):
            _start(b, b)

    def body(g, carry):
        for b in (0, 1):
            j = g * 2 + b

            @pl.when(j < cnt)
            def _(j=j, b=b):
                pltpu.make_async_copy(
                    tab_ref.at[idx_ref.at[pl.ds(0, GCH)]],
                    bufs[b], sems[b]).wait()
                pltpu.sync_copy(bufs[b],
                                g_ref.at[pl.ds((start + j) * GCH, GCH)])

            @pl.when(j + 2 < cnt)
            def _(j=j, b=b):
                _start(j + 2, b)
        return carry

    lax.fori_loop(0, quota // 2, body, 0)


def _quota(nchunk):
    q = -(-nchunk // NW)
    return -(-q // 8) * 8


def _sc_gather(tab, src, e):
    nchunk = e // GCH
    quota = _quota(nchunk)
    mesh = plsc.VectorSubcoreMesh(core_axis_name="c", subcore_axis_name="s")
    kern = pl.kernel(
        functools.partial(_gather_body, nchunk, quota),
        out_type=jax.ShapeDtypeStruct((e, 3 * H), jnp.float32),
        mesh=mesh,
        scratch_types=[
            pltpu.VMEM((quota * GCH,), jnp.int32),
            pltpu.VMEM((GCH, 3 * H), jnp.float32),
            pltpu.VMEM((GCH, 3 * H), jnp.float32),
            pltpu.SemaphoreType.DMA,
            pltpu.SemaphoreType.DMA,
        ],
    )
    return kern(tab, src)


def _edge_body(rbf_ref, aux_ref, g_ref,
               w1_ref, b1_ref, w2_ref, b2_ref, a_ref,
               ms_ref, mv0_ref, mv1_ref, mv2_ref):
    x = rbf_ref[...]
    h = lax.dot_general(x, w1_ref[...], (((1,), (1,)), ((), ())),
                        preferred_element_type=jnp.float32)
    h = _prelu(h + b1_ref[...], a_ref[0])
    fw = lax.dot_general(h, w2_ref[...], (((1,), (1,)), ((), ())),
                         preferred_element_type=jnp.float32)
    fw = _prelu(fw + b2_ref[...], a_ref[1])
    wi = lax.bitcast_convert_type(g_ref[...], jnp.int32)
    gs = lax.bitcast_convert_type(wi << 16, jnp.float32)
    gv = lax.bitcast_convert_type(wi & jnp.int32(-65536), jnp.float32)
    fo = gs * fw
    gate = fo[:, :H]
    ge = fo[:, 2 * H:]
    aux = aux_ref[...]
    rinv = 1.0 / aux[:, 3:4]
    ms_ref[...] = fo[:, H:2 * H]
    for k, mv in ((0, mv0_ref), (1, mv1_ref), (2, mv2_ref)):
        mv[...] = (gv[:, k * H:(k + 1) * H] * gate
                   + (aux[:, k:k + 1] * rinv) * ge)


def _edge_stage(rbf_pad, aux, g, w1p, b1, w2, b2, avec, e):
    grid = e // BE
    full = lambda shape: pl.BlockSpec(shape, lambda i: (0, 0))
    blk = lambda w: pl.BlockSpec((BE, w), lambda i: (i, 0))
    o = jax.ShapeDtypeStruct((e, H), jnp.float32)
    return pl.pallas_call(
        _edge_body,
        grid=(grid,),
        in_specs=[
            blk(32), blk(8), blk(3 * H),
            full((H, 32)), full((1, H)), full((3 * H, H)), full((1, 3 * H)),
            pl.BlockSpec(memory_space=pltpu.SMEM),
        ],
        out_specs=[blk(H), blk(H), blk(H), blk(H)],
        out_shape=[o, o, o, o],
    )(rbf_pad, aux, g, w1p, b1.reshape(1, H), w2, b2.reshape(1, 3 * H), avec)


def _scatter_body(nchunk, n, quota,
                  ms_ref, m0_ref, m1_ref, m2_ref,
                  dst_ref, zeros_ref, out_ref,
                  idx0_ref, idx1_ref, msg0_ref, msg1_ref, w_ref,
                  acc_ref, msem0, msem1, isem0, isem1):
    cid = lax.axis_index("c")
    tid = lax.axis_index("s")
    wid = tid * 2 + cid
    start = wid * quota
    cnt = jnp.clip(nchunk - start, 0, quota)
    zrows = 80
    nzchunk = n // zrows
    ziters = -(-nzchunk // 16)

    mbufs = (msg0_ref, msg1_ref)
    ibufs = (idx0_ref, idx1_ref)
    msems = (msem0, msem1)
    isems = (isem0, isem1)

    for p, msg in enumerate((ms_ref, m0_ref, m1_ref, m2_ref)):
        pltpu.sync_copy(zeros_ref, w_ref)
        for zi in range(ziters):
            c = zi * 16 + tid

            @pl.when(c < nzchunk)
            def _():
                pltpu.sync_copy(w_ref, acc_ref.at[pl.ds(c * zrows, zrows)])
        plsc.subcore_barrier()

        def _start(j, b, msg=msg):
            base = (start + j) * CH
            pltpu.async_copy(msg.at[pl.ds(base, CH)], mbufs[b], msems[b])
            pltpu.async_copy(dst_ref.at[pl.ds(base, CH)], ibufs[b], isems[b])

        for b in (0, 1):
            @pl.when(b < cnt)
            def _(b=b):
                _start(b, b)

        def body(g, carry, msg=msg, _start=_start):
            for b in (0, 1):
                j = g * 2 + b

                @pl.when(j < cnt)
                def _(j=j, b=b):
                    pltpu.make_async_copy(
                        msg.at[pl.ds(0, CH)], mbufs[b], msems[b]).wait()
                    pltpu.make_async_copy(
                        dst_ref.at[pl.ds(0, CH)], ibufs[b], isems[b]).wait()
                    pltpu.sync_copy(mbufs[b], acc_ref.at[ibufs[b]],
                                    add=True)

                @pl.when(j + 2 < cnt)
                def _(j=j, b=b):
                    _start(j + 2, b)
            return carry

        lax.fori_loop(0, quota // 2, body, 0)
        plsc.subcore_barrier()
        obase = (p * 2 + cid) * n
        for zi in range(ziters):
            c = zi * 16 + tid

            @pl.when(c < nzchunk)
            def _():
                pltpu.sync_copy(acc_ref.at[pl.ds(c * zrows, zrows)], w_ref)
                pltpu.sync_copy(w_ref, out_ref.at[pl.ds(obase + c * zrows, zrows)])
        plsc.subcore_barrier()


def _sc_scatter(msgs, dst, zeros, e, n):
    nchunk = e // CH
    quota = _quota(nchunk)
    mesh = plsc.VectorSubcoreMesh(core_axis_name="c", subcore_axis_name="s")
    kern = pl.kernel(
        functools.partial(_scatter_body, nchunk, n, quota),
        out_type=jax.ShapeDtypeStruct((8 * n, H), jnp.float32),
        mesh=mesh,
        scratch_types=[
            pltpu.VMEM((CH,), jnp.int32),
            pltpu.VMEM((CH,), jnp.int32),
            pltpu.VMEM((CH, H), jnp.float32),
            pltpu.VMEM((CH, H), jnp.float32),
            pltpu.VMEM((80, H), jnp.float32),
            pltpu.VMEM_SHARED((n, H), jnp.float32),
            pltpu.SemaphoreType.DMA,
            pltpu.SemaphoreType.DMA,
            pltpu.SemaphoreType.DMA,
            pltpu.SemaphoreType.DMA,
        ],
    )
    return kern(*msgs, dst, zeros)


def _combine_body(*refs):
    base = refs[:4]
    parts = refs[4:-4]
    outs = refs[-4:]
    for k in range(4):
        acc = base[k][...]
        for a in range(len(parts) // 8):
            acc = acc + parts[8 * a + 2 * k][...] + parts[8 * a + 2 * k + 1][...]
        outs[k][...] = acc


def _combine(node_scalar, nv0, nv1, nv2, part_list, n):
    grid = n // BN
    nb = n // BN
    blk = pl.BlockSpec((BN, H), lambda i: (i, 0))

    def pblk(p, c):
        off = (p * 2 + c) * nb
        return pl.BlockSpec((BN, H), lambda i, off=off: (off + i, 0))

    pblks = [pblk(p, c) for p in range(4) for c in range(2)]
    o = jax.ShapeDtypeStruct((n, H), jnp.float32)
    operands = []
    for part in part_list:
        operands.extend([part] * 8)
    return pl.pallas_call(
        _combine_body,
        grid=(grid,),
        in_specs=[blk, blk, blk, blk] + pblks * len(part_list),
        out_specs=[blk, blk, blk, blk],
        out_shape=[o, o, o, o],
    )(node_scalar, nv0, nv1, nv2, *operands)


def kernel(node_scalar, node_vector, che_edge, che_edge_diff, che_edge_dist,
           che_rbf_dist, vdw_edge, vdw_edge_diff, vdw_edge_dist, vdw_rbf_dist,
           che_s_W1, che_s_b1, che_s_a1, che_s_W2, che_s_b2, che_s_a2,
           che_f_W1, che_f_b1, che_f_a1, che_f_W2, che_f_b2, che_f_a2,
           vdw_s_W1, vdw_s_b1, vdw_s_a1, vdw_s_W2, vdw_s_b2, vdw_s_a2,
           vdw_f_W1, vdw_f_b1, vdw_f_a1, vdw_f_W2, vdw_f_b2, vdw_f_a2):
    n = node_scalar.shape[0]
    e = che_edge.shape[0]
    e2 = e // 2
    pad_e2 = NW * _quota(e2 // CH) * CH - e2
    pad_g2 = NW * _quota(e2 // GCH) * GCH - e2

    a_s = jnp.stack([che_s_a1, che_s_a2, vdw_s_a1, vdw_s_a2])
    tab_nv = node_vector.reshape(n, 3 * H)
    tab_che, tab_vdw = _node_mlp(node_scalar, tab_nv, che_s_W1, che_s_b1,
                                 che_s_W2, che_s_b2, vdw_s_W1, vdw_s_b1,
                                 vdw_s_W2, vdw_s_b2, a_s)

    def _pack(tab):
        t16 = tab.astype(jnp.bfloat16)
        t2 = jnp.stack([t16[:, :3 * H], t16[:, 3 * H:]], axis=-1)
        return lax.bitcast_convert_type(t2, jnp.float32)

    a_fc = jnp.stack([che_f_a1, che_f_a2])
    a_fv = jnp.stack([vdw_f_a1, vdw_f_a2])
    ee = che_rbf_dist.shape[1]
    rbf_che = jnp.pad(che_rbf_dist, ((0, 0), (0, 32 - ee)))
    rbf_vdw = jnp.pad(vdw_rbf_dist, ((0, 0), (0, 32 - ee)))
    w1c = jnp.pad(che_f_W1, ((0, 0), (0, 32 - ee)))
    w1v = jnp.pad(vdw_f_W1, ((0, 0), (0, 32 - ee)))
    aux_che = jnp.pad(
        jnp.concatenate([che_edge_diff, che_edge_dist[:, None]], axis=1),
        ((0, 0), (0, 4)))
    aux_vdw = jnp.pad(
        jnp.concatenate([vdw_edge_diff, vdw_edge_dist[:, None]], axis=1),
        ((0, 0), (0, 4)))

    chains = (
        (_pack(tab_che), che_edge, rbf_che, aux_che,
         (w1c, che_f_b1, che_f_W2, che_f_b2, a_fc)),
        (_pack(tab_vdw), vdw_edge, rbf_vdw, aux_vdw,
         (w1v, vdw_f_b1, vdw_f_W2, vdw_f_b2, a_fv)),
    )
    halves = []
    for tab, edge, rbf, aux, fil in chains:
        for h in (0, 1):
            sl = slice(h * e2, (h + 1) * e2)
            src = jnp.pad(edge[sl, 1], (0, pad_g2))
            dst = jnp.pad(edge[sl, 0], (0, pad_e2))
            halves.append((tab, src, dst, rbf[sl], aux[sl], fil))

    gs = [_sc_gather(tab, src, e2) for tab, src, _, _, _, _ in halves]
    msgs = [_edge_stage(rbf, aux, g, *fil, e2)
            for g, (_, _, _, rbf, aux, fil) in zip(gs, halves)]
    zeros = jnp.zeros((80, H), jnp.float32)
    parts = [_sc_scatter(m, dst, zeros, e2, n)
             for m, (_, _, dst, _, _, _) in zip(msgs, halves)]

    nv0 = node_vector[:, 0, :]
    nv1 = node_vector[:, 1, :]
    nv2 = node_vector[:, 2, :]
    os_, ov0, ov1, ov2 = _combine(node_scalar, nv0, nv1, nv2, parts, n)
    return os_, jnp.stack([ov0, ov1, ov2], axis=1)


# speedup vs baseline: 1.0569x; 1.0008x over previous
import functools

import jax
import jax.numpy as jnp
from jax import lax
from jax.experimental import pallas as pl
from jax.experimental.pallas import tpu as pltpu
from jax.experimental.pallas import tpu_sc as plsc

H = 128
CH = 128
NW = 32
BN = 1000
BE = 2000


def _prelu(x, a):
    return jnp.where(x >= 0, x, a * x)


def _node_mlp_body(ns_ref, nv_ref, w1c_ref, b1c_ref, w2c_ref, b2c_ref,
                   w1v_ref, b1v_ref, w2v_ref, b2v_ref, a_ref,
                   outc_ref, outv_ref):
    x = ns_ref[...]
    nv = nv_ref[...]
    for (w1, b1, w2, b2, ia, out) in (
        (w1c_ref, b1c_ref, w2c_ref, b2c_ref, 0, outc_ref),
        (w1v_ref, b1v_ref, w2v_ref, b2v_ref, 2, outv_ref),
    ):
        h = lax.dot_general(x, w1[...], (((1,), (1,)), ((), ())),
                            preferred_element_type=jnp.float32)
        h = _prelu(h + b1[...], a_ref[ia])
        o = lax.dot_general(h, w2[...], (((1,), (1,)), ((), ())),
                            preferred_element_type=jnp.float32)
        out[:, :3 * H] = _prelu(o + b2[...], a_ref[ia + 1])
        out[:, 3 * H:] = nv


def _node_mlp(node_scalar, tab_nv, w1c, b1c, w2c, b2c, w1v, b1v, w2v, b2v,
              avec):
    n = node_scalar.shape[0]
    grid = n // BN
    full = lambda shape: pl.BlockSpec(shape, lambda i: (0, 0))
    return pl.pallas_call(
        _node_mlp_body,
        grid=(grid,),
        in_specs=[
            pl.BlockSpec((BN, H), lambda i: (i, 0)),
            pl.BlockSpec((BN, 3 * H), lambda i: (i, 0)),
            full((H, H)), full((1, H)), full((3 * H, H)), full((1, 3 * H)),
            full((H, H)), full((1, H)), full((3 * H, H)), full((1, 3 * H)),
            pl.BlockSpec(memory_space=pltpu.SMEM),
        ],
        out_specs=[
            pl.BlockSpec((BN, 6 * H), lambda i: (i, 0)),
            pl.BlockSpec((BN, 6 * H), lambda i: (i, 0)),
        ],
        out_shape=[
            jax.ShapeDtypeStruct((n, 6 * H), jnp.float32),
            jax.ShapeDtypeStruct((n, 6 * H), jnp.float32),
        ],
    )(node_scalar, tab_nv, w1c, b1c.reshape(1, H), w2c, b2c.reshape(1, 3 * H),
      w1v, b1v.reshape(1, H), w2v, b2v.reshape(1, 3 * H), avec)


GCH = 128


def _gather_body(nchunk, quota, tab_ref, src_ref, g_ref,
                 idx_ref, buf0_ref, buf1_ref, sem0, sem1, wsem0, wsem1):
    wid = lax.axis_index("s") * 2 + lax.axis_index("c")
    start = wid * quota
    cnt = jnp.clip(nchunk - start, 0, quota)
    bufs = (buf0_ref, buf1_ref)
    sems = (sem0, sem1)
    wsems = (wsem0, wsem1)

    pltpu.sync_copy(src_ref.at[pl.ds(start * GCH, quota * GCH)], idx_ref)

    def _start(j, b):
        idx = idx_ref.at[pl.ds(j * GCH, GCH)]
        pltpu.async_copy(tab_ref.at[idx], bufs[b], sems[b])

    for b in (0, 1):
        @pl.when(b < cnt)
        def _(b=b):
            _start(b, b)

    def body(g, carry):
        for b in (0, 1):
            j = g * 2 + b

            @pl.when(j < cnt)
            def _(j=j, b=b):
                pltpu.make_async_copy(
                    tab_ref.at[idx_ref.at[pl.ds(0, GCH)]],
                    bufs[b], sems[b]).wait()
                pltpu.async_copy(bufs[b],
                                 g_ref.at[pl.ds((start + j) * GCH, GCH)],
                                 wsems[b])
        for b in (0, 1):
            j = g * 2 + b

            @pl.when(j + 2 < cnt)
            def _(j=j, b=b):
                pltpu.make_async_copy(
                    bufs[b], g_ref.at[pl.ds(0, GCH)], wsems[b]).wait()
                _start(j + 2, b)
        return carry

    lax.fori_loop(0, quota // 2, body, 0)
    for b in (0, 1):
        @pl.when(b < cnt)
        def _(b=b):
            pltpu.make_async_copy(
                bufs[b], g_ref.at[pl.ds(0, GCH)], wsems[b]).wait()


def _quota(nchunk):
    q = -(-nchunk // NW)
    return -(-q // 8) * 8


def _sc_gather(tab, src, e):
    nchunk = e // GCH
    quota = _quota(nchunk)
    mesh = plsc.VectorSubcoreMesh(core_axis_name="c", subcore_axis_name="s")
    kern = pl.kernel(
        functools.partial(_gather_body, nchunk, quota),
        out_type=jax.ShapeDtypeStruct((e, 3 * H), jnp.float32),
        mesh=mesh,
        scratch_types=[
            pltpu.VMEM((quota * GCH,), jnp.int32),
            pltpu.VMEM((GCH, 3 * H), jnp.float32),
            pltpu.VMEM((GCH, 3 * H), jnp.float32),
            pltpu.SemaphoreType.DMA,
            pltpu.SemaphoreType.DMA,
            pltpu.SemaphoreType.DMA,
            pltpu.SemaphoreType.DMA,
        ],
    )
    return kern(tab, src)


def _edge_body(rbf_ref, aux_ref, g_ref,
               w1_ref, b1_ref, w2_ref, b2_ref, a_ref,
               ms_ref, mv0_ref, mv1_ref, mv2_ref):
    x = rbf_ref[...]
    h = lax.dot_general(x, w1_ref[...], (((1,), (1,)), ((), ())),
                        preferred_element_type=jnp.float32)
    h = _prelu(h + b1_ref[...], a_ref[0])
    fw = lax.dot_general(h, w2_ref[...], (((1,), (1,)), ((), ())),
                         preferred_element_type=jnp.float32)
    fw = _prelu(fw + b2_ref[...], a_ref[1])
    wi = lax.bitcast_convert_type(g_ref[...], jnp.int32)
    gs = lax.bitcast_convert_type(wi << 16, jnp.float32)
    gv = lax.bitcast_convert_type(wi & jnp.int32(-65536), jnp.float32)
    fo = gs * fw
    gate = fo[:, :H]
    ge = fo[:, 2 * H:]
    aux = aux_ref[...]
    rinv = 1.0 / aux[:, 3:4]
    ms_ref[...] = fo[:, H:2 * H]
    for k, mv in ((0, mv0_ref), (1, mv1_ref), (2, mv2_ref)):
        mv[...] = (gv[:, k * H:(k + 1) * H] * gate
                   + (aux[:, k:k + 1] * rinv) * ge)


def _edge_stage(rbf_pad, aux, g, w1p, b1, w2, b2, avec, e):
    grid = e // BE
    full = lambda shape: pl.BlockSpec(shape, lambda i: (0, 0))
    blk = lambda w: pl.BlockSpec((BE, w), lambda i: (i, 0))
    o = jax.ShapeDtypeStruct((e, H), jnp.float32)
    return pl.pallas_call(
        _edge_body,
        grid=(grid,),
        in_specs=[
            blk(32), blk(8), blk(3 * H),
            full((H, 32)), full((1, H)), full((3 * H, H)), full((1, 3 * H)),
            pl.BlockSpec(memory_space=pltpu.SMEM),
        ],
        out_specs=[blk(H), blk(H), blk(H), blk(H)],
        out_shape=[o, o, o, o],
    )(rbf_pad, aux, g, w1p, b1.reshape(1, H), w2, b2.reshape(1, 3 * H), avec)


def _scatter_body(nchunk, n, quota,
                  ms_ref, m0_ref, m1_ref, m2_ref,
                  dst_ref, zeros_ref, out_ref,
                  idx0_ref, idx1_ref, msg0_ref, msg1_ref, w_ref,
                  acc_ref, msem0, msem1, isem0, isem1):
    cid = lax.axis_index("c")
    tid = lax.axis_index("s")
    wid = tid * 2 + cid
    start = wid * quota
    cnt = jnp.clip(nchunk - start, 0, quota)
    zrows = 80
    nzchunk = n // zrows
    ziters = -(-nzchunk // 16)

    mbufs = (msg0_ref, msg1_ref)
    ibufs = (idx0_ref, idx1_ref)
    msems = (msem0, msem1)
    isems = (isem0, isem1)

    for p, msg in enumerate((ms_ref, m0_ref, m1_ref, m2_ref)):
        pltpu.sync_copy(zeros_ref, w_ref)
        for zi in range(ziters):
            c = zi * 16 + tid

            @pl.when(c < nzchunk)
            def _():
                pltpu.sync_copy(w_ref, acc_ref.at[pl.ds(c * zrows, zrows)])
        plsc.subcore_barrier()

        def _start(j, b, msg=msg):
            base = (start + j) * CH
            pltpu.async_copy(msg.at[pl.ds(base, CH)], mbufs[b], msems[b])
            pltpu.async_copy(dst_ref.at[pl.ds(base, CH)], ibufs[b], isems[b])

        for b in (0, 1):
            @pl.when(b < cnt)
            def _(b=b):
                _start(b, b)

        def body(g, carry, msg=msg, _start=_start):
            for b in (0, 1):
                j = g * 2 + b

                @pl.when(j < cnt)
                def _(j=j, b=b):
                    pltpu.make_async_copy(
                        msg.at[pl.ds(0, CH)], mbufs[b], msems[b]).wait()
                    pltpu.make_async_copy(
                        dst_ref.at[pl.ds(0, CH)], ibufs[b], isems[b]).wait()
                    pltpu.sync_copy(mbufs[b], acc_ref.at[ibufs[b]],
                                    add=True)

                @pl.when(j + 2 < cnt)
                def _(j=j, b=b):
                    _start(j + 2, b)
            return carry

        lax.fori_loop(0, quota // 2, body, 0)
        plsc.subcore_barrier()
        obase = (p * 2 + cid) * n
        for zi in range(ziters):
            c = zi * 16 + tid

            @pl.when(c < nzchunk)
            def _():
                pltpu.sync_copy(acc_ref.at[pl.ds(c * zrows, zrows)], w_ref)
                pltpu.sync_copy(w_ref, out_ref.at[pl.ds(obase + c * zrows, zrows)])
        plsc.subcore_barrier()


def _sc_scatter(msgs, dst, zeros, e, n):
    nchunk = e // CH
    quota = _quota(nchunk)
    mesh = plsc.VectorSubcoreMesh(core_axis_name="c", subcore_axis_name="s")
    kern = pl.kernel(
        functools.partial(_scatter_body, nchunk, n, quota),
        out_type=jax.ShapeDtypeStruct((8 * n, H), jnp.float32),
        mesh=mesh,
        scratch_types=[
            pltpu.VMEM((CH,), jnp.int32),
            pltpu.VMEM((CH,), jnp.int32),
            pltpu.VMEM((CH, H), jnp.float32),
            pltpu.VMEM((CH, H), jnp.float32),
            pltpu.VMEM((80, H), jnp.float32),
            pltpu.VMEM_SHARED((n, H), jnp.float32),
            pltpu.SemaphoreType.DMA,
            pltpu.SemaphoreType.DMA,
            pltpu.SemaphoreType.DMA,
            pltpu.SemaphoreType.DMA,
        ],
    )
    return kern(*msgs, dst, zeros)


def _combine_body(*refs):
    base = refs[:4]
    parts = refs[4:-4]
    outs = refs[-4:]
    for k in range(4):
        acc = base[k][...]
        for a in range(len(parts) // 8):
            acc = acc + parts[8 * a + 2 * k][...] + parts[8 * a + 2 * k + 1][...]
        outs[k][...] = acc


def _combine(node_scalar, nv0, nv1, nv2, part_list, n):
    grid = n // BN
    nb = n // BN
    blk = pl.BlockSpec((BN, H), lambda i: (i, 0))

    def pblk(p, c):
        off = (p * 2 + c) * nb
        return pl.BlockSpec((BN, H), lambda i, off=off: (off + i, 0))

    pblks = [pblk(p, c) for p in range(4) for c in range(2)]
    o = jax.ShapeDtypeStruct((n, H), jnp.float32)
    operands = []
    for part in part_list:
        operands.extend([part] * 8)
    return pl.pallas_call(
        _combine_body,
        grid=(grid,),
        in_specs=[blk, blk, blk, blk] + pblks * len(part_list),
        out_specs=[blk, blk, blk, blk],
        out_shape=[o, o, o, o],
    )(node_scalar, nv0, nv1, nv2, *operands)


def kernel(node_scalar, node_vector, che_edge, che_edge_diff, che_edge_dist,
           che_rbf_dist, vdw_edge, vdw_edge_diff, vdw_edge_dist, vdw_rbf_dist,
           che_s_W1, che_s_b1, che_s_a1, che_s_W2, che_s_b2, che_s_a2,
           che_f_W1, che_f_b1, che_f_a1, che_f_W2, che_f_b2, che_f_a2,
           vdw_s_W1, vdw_s_b1, vdw_s_a1, vdw_s_W2, vdw_s_b2, vdw_s_a2,
           vdw_f_W1, vdw_f_b1, vdw_f_a1, vdw_f_W2, vdw_f_b2, vdw_f_a2):
    n = node_scalar.shape[0]
    e = che_edge.shape[0]
    e2 = e
    pad_e2 = NW * _quota(e2 // CH) * CH - e2
    pad_g2 = NW * _quota(e2 // GCH) * GCH - e2

    a_s = jnp.stack([che_s_a1, che_s_a2, vdw_s_a1, vdw_s_a2])
    tab_nv = node_vector.reshape(n, 3 * H)
    tab_che, tab_vdw = _node_mlp(node_scalar, tab_nv, che_s_W1, che_s_b1,
                                 che_s_W2, che_s_b2, vdw_s_W1, vdw_s_b1,
                                 vdw_s_W2, vdw_s_b2, a_s)

    def _pack(tab):
        t16 = tab.astype(jnp.bfloat16)
        t2 = jnp.stack([t16[:, :3 * H], t16[:, 3 * H:]], axis=-1)
        return lax.bitcast_convert_type(t2, jnp.float32)

    a_fc = jnp.stack([che_f_a1, che_f_a2])
    a_fv = jnp.stack([vdw_f_a1, vdw_f_a2])
    ee = che_rbf_dist.shape[1]
    rbf_che = jnp.pad(che_rbf_dist, ((0, 0), (0, 32 - ee)))
    rbf_vdw = jnp.pad(vdw_rbf_dist, ((0, 0), (0, 32 - ee)))
    w1c = jnp.pad(che_f_W1, ((0, 0), (0, 32 - ee)))
    w1v = jnp.pad(vdw_f_W1, ((0, 0), (0, 32 - ee)))
    aux_che = jnp.pad(
        jnp.concatenate([che_edge_diff, che_edge_dist[:, None]], axis=1),
        ((0, 0), (0, 4)))
    aux_vdw = jnp.pad(
        jnp.concatenate([vdw_edge_diff, vdw_edge_dist[:, None]], axis=1),
        ((0, 0), (0, 4)))

    chains = (
        (_pack(tab_che), che_edge, rbf_che, aux_che,
         (w1c, che_f_b1, che_f_W2, che_f_b2, a_fc)),
        (_pack(tab_vdw), vdw_edge, rbf_vdw, aux_vdw,
         (w1v, vdw_f_b1, vdw_f_W2, vdw_f_b2, a_fv)),
    )
    halves = []
    for tab, edge, rbf, aux, fil in chains:
        src = jnp.pad(edge[:, 1], (0, pad_g2))
        dst = jnp.pad(edge[:, 0], (0, pad_e2))
        halves.append((tab, src, dst, rbf, aux, fil))

    gs = [_sc_gather(tab, src, e2) for tab, src, _, _, _, _ in halves]
    msgs = [_edge_stage(rbf, aux, g, *fil, e2)
            for g, (_, _, _, rbf, aux, fil) in zip(gs, halves)]
    zeros = jnp.zeros((80, H), jnp.float32)
    parts = [_sc_scatter(m, dst, zeros, e2, n)
             for m, (_, _, dst, _, _, _) in zip(msgs, halves)]

    nv0 = node_vector[:, 0, :]
    nv1 = node_vector[:, 1, :]
    nv2 = node_vector[:, 2, :]
    os_, ov0, ov1, ov2 = _combine(node_scalar, nv0, nv1, nv2, parts, n)
    return os_, jnp.stack([ov0, ov1, ov2], axis=1)


# speedup vs baseline: 1.1010x; 1.0418x over previous
import functools

import jax
import jax.numpy as jnp
from jax import lax
from jax.experimental import pallas as pl
from jax.experimental.pallas import tpu as pltpu
from jax.experimental.pallas import tpu_sc as plsc

H = 128
CH = 128
NW = 32
BN = 1000
BE = 2000


def _prelu(x, a):
    return jnp.where(x >= 0, x, a * x)


def _node_mlp_body(ns_ref, nv_ref, w1c_ref, b1c_ref, w2c_ref, b2c_ref,
                   w1v_ref, b1v_ref, w2v_ref, b2v_ref, a_ref,
                   outc_ref, outv_ref):
    x = ns_ref[...]
    nv = nv_ref[...]
    for (w1, b1, w2, b2, ia, out) in (
        (w1c_ref, b1c_ref, w2c_ref, b2c_ref, 0, outc_ref),
        (w1v_ref, b1v_ref, w2v_ref, b2v_ref, 2, outv_ref),
    ):
        h = lax.dot_general(x, w1[...], (((1,), (1,)), ((), ())),
                            preferred_element_type=jnp.float32)
        h = _prelu(h + b1[...], a_ref[ia])
        o = lax.dot_general(h, w2[...], (((1,), (1,)), ((), ())),
                            preferred_element_type=jnp.float32)
        out[:, :3 * H] = _prelu(o + b2[...], a_ref[ia + 1])
        out[:, 3 * H:] = nv


def _node_mlp(node_scalar, tab_nv, w1c, b1c, w2c, b2c, w1v, b1v, w2v, b2v,
              avec):
    n = node_scalar.shape[0]
    grid = n // BN
    full = lambda shape: pl.BlockSpec(shape, lambda i: (0, 0))
    return pl.pallas_call(
        _node_mlp_body,
        grid=(grid,),
        in_specs=[
            pl.BlockSpec((BN, H), lambda i: (i, 0)),
            pl.BlockSpec((BN, 3 * H), lambda i: (i, 0)),
            full((H, H)), full((1, H)), full((3 * H, H)), full((1, 3 * H)),
            full((H, H)), full((1, H)), full((3 * H, H)), full((1, 3 * H)),
            pl.BlockSpec(memory_space=pltpu.SMEM),
        ],
        out_specs=[
            pl.BlockSpec((BN, 6 * H), lambda i: (i, 0)),
            pl.BlockSpec((BN, 6 * H), lambda i: (i, 0)),
        ],
        out_shape=[
            jax.ShapeDtypeStruct((n, 6 * H), jnp.float32),
            jax.ShapeDtypeStruct((n, 6 * H), jnp.float32),
        ],
    )(node_scalar, tab_nv, w1c, b1c.reshape(1, H), w2c, b2c.reshape(1, 3 * H),
      w1v, b1v.reshape(1, H), w2v, b2v.reshape(1, 3 * H), avec)


GCH = 128


def _gather_body(nchunk, quota, tab_ref, src_ref, g_ref,
                 idx_ref, buf0_ref, buf1_ref, sem0, sem1, wsem0, wsem1):
    wid = lax.axis_index("s") * 2 + lax.axis_index("c")
    start = wid * quota
    cnt = jnp.clip(nchunk - start, 0, quota)
    bufs = (buf0_ref, buf1_ref)
    sems = (sem0, sem1)
    wsems = (wsem0, wsem1)

    pltpu.sync_copy(src_ref.at[pl.ds(start * GCH, quota * GCH)], idx_ref)

    def _start(j, b):
        idx = idx_ref.at[pl.ds(j * GCH, GCH)]
        pltpu.async_copy(tab_ref.at[idx], bufs[b], sems[b])

    for b in (0, 1):
        @pl.when(b < cnt)
        def _(b=b):
            _start(b, b)

    def body(g, carry):
        for b in (0, 1):
            j = g * 2 + b

            @pl.when(j < cnt)
            def _(j=j, b=b):
                pltpu.make_async_copy(
                    tab_ref.at[idx_ref.at[pl.ds(0, GCH)]],
                    bufs[b], sems[b]).wait()
                pltpu.async_copy(bufs[b],
                                 g_ref.at[pl.ds((start + j) * GCH, GCH)],
                                 wsems[b])
        for b in (0, 1):
            j = g * 2 + b

            @pl.when(j + 2 < cnt)
            def _(j=j, b=b):
                pltpu.make_async_copy(
                    bufs[b], g_ref.at[pl.ds(0, GCH)], wsems[b]).wait()
                _start(j + 2, b)
        return carry

    lax.fori_loop(0, quota // 2, body, 0)
    for b in (0, 1):
        @pl.when(b < cnt)
        def _(b=b):
            pltpu.make_async_copy(
                bufs[b], g_ref.at[pl.ds(0, GCH)], wsems[b]).wait()


def _quota(nchunk):
    q = -(-nchunk // NW)
    return -(-q // 8) * 8


def _sc_gather(tab, src, e):
    nchunk = e // GCH
    quota = _quota(nchunk)
    mesh = plsc.VectorSubcoreMesh(core_axis_name="c", subcore_axis_name="s")
    kern = pl.kernel(
        functools.partial(_gather_body, nchunk, quota),
        out_type=jax.ShapeDtypeStruct((e, 3 * H), jnp.float32),
        mesh=mesh,
        scratch_types=[
            pltpu.VMEM((quota * GCH,), jnp.int32),
            pltpu.VMEM((GCH, 3 * H), jnp.float32),
            pltpu.VMEM((GCH, 3 * H), jnp.float32),
            pltpu.SemaphoreType.DMA,
            pltpu.SemaphoreType.DMA,
            pltpu.SemaphoreType.DMA,
            pltpu.SemaphoreType.DMA,
        ],
    )
    return kern(tab, src)


def _edge_body(rbf_ref, dd_ref, dist_ref, g_ref,
               w1_ref, b1_ref, w2_ref, b2_ref, a_ref,
               ms_ref, mv0_ref, mv1_ref, mv2_ref):
    x = rbf_ref[...]
    h = lax.dot_general(x, w1_ref[...], (((1,), (1,)), ((), ())),
                        preferred_element_type=jnp.float32)
    h = _prelu(h + b1_ref[...], a_ref[0])
    fw = lax.dot_general(h, w2_ref[...], (((1,), (1,)), ((), ())),
                         preferred_element_type=jnp.float32)
    fw = _prelu(fw + b2_ref[...], a_ref[1])
    wi = lax.bitcast_convert_type(g_ref[...], jnp.int32)
    gs = lax.bitcast_convert_type(wi << 16, jnp.float32)
    gv = lax.bitcast_convert_type(wi & jnp.int32(-65536), jnp.float32)
    fo = gs * fw
    gate = fo[:, :H]
    ge = fo[:, 2 * H:]
    rinv = 1.0 / dist_ref[...]
    dd = dd_ref[...]
    ms_ref[...] = fo[:, H:2 * H]
    for k, mv in ((0, mv0_ref), (1, mv1_ref), (2, mv2_ref)):
        mv[...] = (gv[:, k * H:(k + 1) * H] * gate
                   + (dd[:, k:k + 1] * rinv) * ge)


def _edge_stage(rbf, dd, dist, g, w1, b1, w2, b2, avec, e):
    grid = e // BE
    ee = rbf.shape[1]
    full = lambda shape: pl.BlockSpec(shape, lambda i: (0, 0))
    blk = lambda w: pl.BlockSpec((BE, w), lambda i: (i, 0))
    o = jax.ShapeDtypeStruct((e, H), jnp.float32)
    return pl.pallas_call(
        _edge_body,
        grid=(grid,),
        in_specs=[
            blk(ee), blk(3), blk(1), blk(3 * H),
            full((H, ee)), full((1, H)), full((3 * H, H)), full((1, 3 * H)),
            pl.BlockSpec(memory_space=pltpu.SMEM),
        ],
        out_specs=[blk(H), blk(H), blk(H), blk(H)],
        out_shape=[o, o, o, o],
    )(rbf, dd, dist, g, w1, b1.reshape(1, H), w2, b2.reshape(1, 3 * H), avec)


def _scatter_body(nchunk, n, quota,
                  ms_ref, m0_ref, m1_ref, m2_ref,
                  dst_ref, zeros_ref, out_ref,
                  idx0_ref, idx1_ref, msg0_ref, msg1_ref, w_ref,
                  acc_ref, msem0, msem1, isem0, isem1):
    cid = lax.axis_index("c")
    tid = lax.axis_index("s")
    wid = tid * 2 + cid
    start = wid * quota
    cnt = jnp.clip(nchunk - start, 0, quota)
    zrows = 80
    nzchunk = n // zrows
    ziters = -(-nzchunk // 16)

    mbufs = (msg0_ref, msg1_ref)
    ibufs = (idx0_ref, idx1_ref)
    msems = (msem0, msem1)
    isems = (isem0, isem1)

    for p, msg in enumerate((ms_ref, m0_ref, m1_ref, m2_ref)):
        pltpu.sync_copy(zeros_ref, w_ref)
        for zi in range(ziters):
            c = zi * 16 + tid

            @pl.when(c < nzchunk)
            def _():
                pltpu.sync_copy(w_ref, acc_ref.at[pl.ds(c * zrows, zrows)])
        plsc.subcore_barrier()

        def _start(j, b, msg=msg):
            base = (start + j) * CH
            pltpu.async_copy(msg.at[pl.ds(base, CH)], mbufs[b], msems[b])
            pltpu.async_copy(dst_ref.at[pl.ds(base, CH)], ibufs[b], isems[b])

        for b in (0, 1):
            @pl.when(b < cnt)
            def _(b=b):
                _start(b, b)

        def body(g, carry, msg=msg, _start=_start):
            for b in (0, 1):
                j = g * 2 + b

                @pl.when(j < cnt)
                def _(j=j, b=b):
                    pltpu.make_async_copy(
                        msg.at[pl.ds(0, CH)], mbufs[b], msems[b]).wait()
                    pltpu.make_async_copy(
                        dst_ref.at[pl.ds(0, CH)], ibufs[b], isems[b]).wait()
                    pltpu.sync_copy(mbufs[b], acc_ref.at[ibufs[b]],
                                    add=True)

                @pl.when(j + 2 < cnt)
                def _(j=j, b=b):
                    _start(j + 2, b)
            return carry

        lax.fori_loop(0, quota // 2, body, 0)
        plsc.subcore_barrier()
        obase = (p * 2 + cid) * n
        for zi in range(ziters):
            c = zi * 16 + tid

            @pl.when(c < nzchunk)
            def _():
                pltpu.sync_copy(acc_ref.at[pl.ds(c * zrows, zrows)], w_ref)
                pltpu.sync_copy(w_ref, out_ref.at[pl.ds(obase + c * zrows, zrows)])
        plsc.subcore_barrier()


def _sc_scatter(msgs, dst, zeros, e, n):
    nchunk = e // CH
    quota = _quota(nchunk)
    mesh = plsc.VectorSubcoreMesh(core_axis_name="c", subcore_axis_name="s")
    kern = pl.kernel(
        functools.partial(_scatter_body, nchunk, n, quota),
        out_type=jax.ShapeDtypeStruct((8 * n, H), jnp.float32),
        mesh=mesh,
        scratch_types=[
            pltpu.VMEM((CH,), jnp.int32),
            pltpu.VMEM((CH,), jnp.int32),
            pltpu.VMEM((CH, H), jnp.float32),
            pltpu.VMEM((CH, H), jnp.float32),
            pltpu.VMEM((80, H), jnp.float32),
            pltpu.VMEM_SHARED((n, H), jnp.float32),
            pltpu.SemaphoreType.DMA,
            pltpu.SemaphoreType.DMA,
            pltpu.SemaphoreType.DMA,
            pltpu.SemaphoreType.DMA,
        ],
    )
    return kern(*msgs, dst, zeros)


def _combine_body(*refs):
    base = refs[:4]
    parts = refs[4:-4]
    outs = refs[-4:]
    for k in range(4):
        acc = base[k][...]
        for a in range(len(parts) // 8):
            acc = acc + parts[8 * a + 2 * k][...] + parts[8 * a + 2 * k + 1][...]
        outs[k][...] = acc


def _combine(node_scalar, nv0, nv1, nv2, part_list, n):
    grid = n // BN
    nb = n // BN
    blk = pl.BlockSpec((BN, H), lambda i: (i, 0))

    def pblk(p, c):
        off = (p * 2 + c) * nb
        return pl.BlockSpec((BN, H), lambda i, off=off: (off + i, 0))

    pblks = [pblk(p, c) for p in range(4) for c in range(2)]
    o = jax.ShapeDtypeStruct((n, H), jnp.float32)
    operands = []
    for part in part_list:
        operands.extend([part] * 8)
    return pl.pallas_call(
        _combine_body,
        grid=(grid,),
        in_specs=[blk, blk, blk, blk] + pblks * len(part_list),
        out_specs=[blk, blk, blk, blk],
        out_shape=[o, o, o, o],
    )(node_scalar, nv0, nv1, nv2, *operands)


def kernel(node_scalar, node_vector, che_edge, che_edge_diff, che_edge_dist,
           che_rbf_dist, vdw_edge, vdw_edge_diff, vdw_edge_dist, vdw_rbf_dist,
           che_s_W1, che_s_b1, che_s_a1, che_s_W2, che_s_b2, che_s_a2,
           che_f_W1, che_f_b1, che_f_a1, che_f_W2, che_f_b2, che_f_a2,
           vdw_s_W1, vdw_s_b1, vdw_s_a1, vdw_s_W2, vdw_s_b2, vdw_s_a2,
           vdw_f_W1, vdw_f_b1, vdw_f_a1, vdw_f_W2, vdw_f_b2, vdw_f_a2):
    n = node_scalar.shape[0]
    e = che_edge.shape[0]
    e2 = e
    pad_e2 = NW * _quota(e2 // CH) * CH - e2
    pad_g2 = NW * _quota(e2 // GCH) * GCH - e2

    a_s = jnp.stack([che_s_a1, che_s_a2, vdw_s_a1, vdw_s_a2])
    tab_nv = node_vector.reshape(n, 3 * H)
    tab_che, tab_vdw = _node_mlp(node_scalar, tab_nv, che_s_W1, che_s_b1,
                                 che_s_W2, che_s_b2, vdw_s_W1, vdw_s_b1,
                                 vdw_s_W2, vdw_s_b2, a_s)

    def _pack(tab):
        t16 = tab.astype(jnp.bfloat16)
        t2 = jnp.stack([t16[:, :3 * H], t16[:, 3 * H:]], axis=-1)
        return lax.bitcast_convert_type(t2, jnp.float32)

    a_fc = jnp.stack([che_f_a1, che_f_a2])
    a_fv = jnp.stack([vdw_f_a1, vdw_f_a2])

    chains = (
        (_pack(tab_che), che_edge, che_rbf_dist, che_edge_diff,
         che_edge_dist.reshape(e, 1),
         (che_f_W1, che_f_b1, che_f_W2, che_f_b2, a_fc)),
        (_pack(tab_vdw), vdw_edge, vdw_rbf_dist, vdw_edge_diff,
         vdw_edge_dist.reshape(e, 1),
         (vdw_f_W1, vdw_f_b1, vdw_f_W2, vdw_f_b2, a_fv)),
    )
    halves = []
    for tab, edge, rbf, dd, dist, fil in chains:
        src = jnp.pad(edge[:, 1], (0, pad_g2))
        dst = jnp.pad(edge[:, 0], (0, pad_e2))
        halves.append((tab, src, dst, rbf, dd, dist, fil))

    gs = [_sc_gather(tab, src, e2) for tab, src, _, _, _, _, _ in halves]
    msgs = [_edge_stage(rbf, dd, dist, g, *fil, e2)
            for g, (_, _, _, rbf, dd, dist, fil) in zip(gs, halves)]
    zeros = jnp.zeros((80, H), jnp.float32)
    parts = [_sc_scatter(m, dst, zeros, e2, n)
             for m, (_, _, dst, _, _, _, _) in zip(msgs, halves)]

    nv0 = node_vector[:, 0, :]
    nv1 = node_vector[:, 1, :]
    nv2 = node_vector[:, 2, :]
    os_, ov0, ov1, ov2 = _combine(node_scalar, nv0, nv1, nv2, parts, n)
    return os_, jnp.stack([ov0, ov1, ov2], axis=1)
